# baseline (speedup 1.0000x reference)
import jax
import jax.numpy as jnp
import numpy as np
from jax import lax
from jax.experimental import pallas as pl
from jax.experimental.pallas import tpu as pltpu

NUM_BUCKETS = 32
MAX_DISTANCE = 128
NUM_HEADS = 16
S = 2048
SKEW = 128
ROW_BLOCK = 64
LINE_LEN = 3968
LINE_PAD = LINE_LEN + SKEW


def _compute_line():
    k = lax.broadcasted_iota(jnp.int32, (NUM_HEADS, LINE_PAD), 1)
    d = k - (S - 1)
    nb = NUM_BUCKETS // 2
    rel_buckets = (d > 0).astype(jnp.int32) * nb
    ad = jnp.abs(d)
    max_exact = nb // 2
    is_small = ad < max_exact
    rp_f = jnp.maximum(ad, 1).astype(jnp.float32)
    large = max_exact + (
        jnp.log(rp_f / max_exact) / np.log(MAX_DISTANCE / max_exact) * (nb - max_exact)
    ).astype(jnp.int32)
    large = jnp.minimum(large, jnp.full_like(large, nb - 1))
    return rel_buckets + jnp.where(is_small, ad, large)


def _fused_kernel(table_ref, out_ref, skew_ref, sem):
    bucket = _compute_line()
    half_w = LINE_PAD // 2
    left = jnp.zeros((NUM_HEADS, half_w), jnp.float32)
    right = jnp.zeros((NUM_HEADS, half_w), jnp.float32)
    b_left = bucket[:, :half_w]
    b_right = bucket[:, half_w:]
    for b in range(NUM_BUCKETS // 2):
        val = table_ref[b, :][:, None]
        left = jnp.where(b_left == b, val, left)
    for b in range(NUM_BUCKETS // 2, NUM_BUCKETS):
        val = table_ref[b, :][:, None]
        right = jnp.where(b_right == b, val, right)
    line = jnp.concatenate([left, right], axis=1)

    copies = []

    def fire(rstart, nrows, off, out_row0):
        cp = pltpu.make_async_copy(
            skew_ref.at[:, pl.ds(rstart, nrows), pl.ds(off, S)],
            out_ref.at[:, pl.ds(out_row0, nrows), :],
            sem,
        )
        cp.start()
        copies.append(cp)

    for half in range(2):
        for q in range(ROW_BLOCK // 8):
            for r in range(half * ROW_BLOCK + q * 8, half * ROW_BLOCK + q * 8 + 8):
                sh = (SKEW - 1) - r
                skew_ref[:, r, :] = line[:, sh : sh + LINE_LEN]
            if half == 0:
                fire(q * 8, 8, S - SKEW, q * 8)
        for gg in range(1 - half, S // SKEW):
            g = 2 * gg + half
            off = (S - SKEW) - SKEW * gg
            fire(half * ROW_BLOCK, ROW_BLOCK, off, g * ROW_BLOCK)
    for cp in copies:
        cp.wait()


def kernel(input_ids, attention_mask, bias_table):
    del input_ids, attention_mask
    out = pl.pallas_call(
        _fused_kernel,
        in_specs=[pl.BlockSpec((NUM_BUCKETS, NUM_HEADS), lambda: (0, 0))],
        out_specs=pl.BlockSpec(memory_space=pl.ANY),
        out_shape=jax.ShapeDtypeStruct((NUM_HEADS, S, S), jnp.float32),
        scratch_shapes=[
            pltpu.VMEM((NUM_HEADS, SKEW, LINE_LEN), jnp.float32),
            pltpu.SemaphoreType.DMA,
        ],
    )(bias_table)
    return out[None]
